# baseline (speedup 1.0000x reference)
import functools

import jax
import jax.numpy as jnp
from jax import lax
from jax.experimental import pallas as pl
from jax.experimental.pallas import tpu as pltpu
from jax.experimental.pallas import tpu_sc as plsc

BATCH = 16384
EMBED = 64
NUM_CORES = 2
NUM_SUBCORES = 16
NW = NUM_CORES * NUM_SUBCORES
B_PER_W = BATCH // NW
CHUNK = 128
NCH = B_PER_W // CHUNK


def _mesh():
    return plsc.VectorSubcoreMesh(core_axis_name="c", subcore_axis_name="s")


def _gather_body(n_slabs, idx_hbm, tab, outs, idx_v, rows_v, sem):
    wid = lax.axis_index("s") * NUM_CORES + lax.axis_index("c")
    base = wid * B_PER_W

    pltpu.sync_copy(idx_hbm.at[wid], idx_v)

    copies = []
    for k in range(n_slabs):
        for c in range(NCH):
            copies.append(
                pltpu.async_copy(
                    tab.at[idx_v.at[k * NCH + c]],
                    rows_v.at[k, pl.ds(c * CHUNK, CHUNK)],
                    sem,
                )
            )
    for cp in copies:
        cp.wait()
    for k in range(n_slabs):
        pltpu.sync_copy(rows_v.at[k], outs[k].at[pl.ds(base, B_PER_W)])


def _user_body(idx_hbm, tab, out_u, idx_v, rows_v, sem):
    _gather_body(1, idx_hbm, tab, (out_u,), idx_v, rows_v, sem)


def _item_body(idx_hbm, tab, out_p, out_n, idx_v, rows_v, sem):
    _gather_body(2, idx_hbm, tab, (out_p, out_n), idx_v, rows_v, sem)


@jax.jit
def _sbpr(idx_u, idx_pn, embed_user, embed_item):
    out = jax.ShapeDtypeStruct((BATCH, EMBED), jnp.float32)
    params = pltpu.CompilerParams(use_tc_tiling_on_sc=False)
    out_u = pl.kernel(
        _user_body,
        out_type=(out,),
        mesh=_mesh(),
        scratch_types=[
            pltpu.VMEM((NCH, CHUNK), jnp.int32),
            pltpu.VMEM((1, B_PER_W, EMBED), jnp.float32),
            pltpu.SemaphoreType.DMA,
        ],
        compiler_params=params,
    )(idx_u, embed_user)[0]
    out_p, out_n = pl.kernel(
        _item_body,
        out_type=(out, out),
        mesh=_mesh(),
        scratch_types=[
            pltpu.VMEM((2 * NCH, CHUNK), jnp.int32),
            pltpu.VMEM((2, B_PER_W, EMBED), jnp.float32),
            pltpu.SemaphoreType.DMA,
        ],
        compiler_params=params,
    )(idx_pn, embed_item)
    return out_u, out_p, out_n


def kernel(batch_user, batch_pos_item, batch_neg_item, embed_user, embed_item):
    idx_u = batch_user.reshape(NW, NCH, CHUNK)
    idx_pn = (
        jnp.stack([batch_pos_item, batch_neg_item])
        .reshape(2, NW, NCH, CHUNK)
        .transpose(1, 0, 2, 3)
        .reshape(NW, 2 * NCH, CHUNK)
    )
    return _sbpr(idx_u, idx_pn, embed_user, embed_item)


# speedup vs baseline: 1.0026x; 1.0026x over previous
import functools

import jax
import jax.numpy as jnp
from jax import lax
from jax.experimental import pallas as pl
from jax.experimental.pallas import tpu as pltpu
from jax.experimental.pallas import tpu_sc as plsc

BATCH = 16384
EMBED = 64
NUM_CORES = 2
NUM_SUBCORES = 16
NW = NUM_CORES * NUM_SUBCORES
B_PER_W = BATCH // NW
CHUNK = 128
NCH = B_PER_W // CHUNK


def _mesh():
    return plsc.VectorSubcoreMesh(core_axis_name="c", subcore_axis_name="s")


def _gather_body(n_slabs, idx_hbm, tab, outs, idx_v, rows_v, sem):
    wid = lax.axis_index("s") * NUM_CORES + lax.axis_index("c")
    base = wid * B_PER_W

    pltpu.sync_copy(idx_hbm.at[wid], idx_v)

    copies = []
    for k in range(n_slabs):
        for c in range(NCH):
            copies.append(
                pltpu.async_copy(
                    tab.at[idx_v.at[k * NCH + c]],
                    rows_v.at[k, pl.ds(c * CHUNK, CHUNK)],
                    sem,
                )
            )
    for cp in copies:
        cp.wait()
    for k in range(n_slabs):
        pltpu.sync_copy(rows_v.at[k], outs[k].at[pl.ds(base, B_PER_W)])


def _user_body(idx_hbm, tab, out_u, idx_v, rows_v, sem):
    _gather_body(1, idx_hbm, tab, (out_u,), idx_v, rows_v, sem)


def _item_body(idx_hbm, tab, out_p, out_n, idx_v, rows_v, sem):
    _gather_body(2, idx_hbm, tab, (out_p, out_n), idx_v, rows_v, sem)


@jax.jit
def _sbpr(idx_u, idx_pn, embed_user, embed_item):
    out = jax.ShapeDtypeStruct((BATCH, EMBED), jnp.float32)
    params = pltpu.CompilerParams(use_tc_tiling_on_sc=False, skip_device_barrier=True)
    out_u = pl.kernel(
        _user_body,
        out_type=(out,),
        mesh=_mesh(),
        scratch_types=[
            pltpu.VMEM((NCH, CHUNK), jnp.int32),
            pltpu.VMEM((1, B_PER_W, EMBED), jnp.float32),
            pltpu.SemaphoreType.DMA,
        ],
        compiler_params=params,
    )(idx_u, embed_user)[0]
    out_p, out_n = pl.kernel(
        _item_body,
        out_type=(out, out),
        mesh=_mesh(),
        scratch_types=[
            pltpu.VMEM((2 * NCH, CHUNK), jnp.int32),
            pltpu.VMEM((2, B_PER_W, EMBED), jnp.float32),
            pltpu.SemaphoreType.DMA,
        ],
        compiler_params=params,
    )(idx_pn, embed_item)
    return out_u, out_p, out_n


def kernel(batch_user, batch_pos_item, batch_neg_item, embed_user, embed_item):
    idx_u = batch_user.reshape(NW, NCH, CHUNK)
    idx_pn = (
        jnp.stack([batch_pos_item, batch_neg_item])
        .reshape(2, NW, NCH, CHUNK)
        .transpose(1, 0, 2, 3)
        .reshape(NW, 2 * NCH, CHUNK)
    )
    return _sbpr(idx_u, idx_pn, embed_user, embed_item)


# speedup vs baseline: 1.5781x; 1.5740x over previous
import functools

import jax
import jax.numpy as jnp
from jax import lax
from jax.experimental import pallas as pl
from jax.experimental.pallas import tpu as pltpu
from jax.experimental.pallas import tpu_sc as plsc

BATCH = 16384
EMBED = 64
NUM_CORES = 2
NUM_SUBCORES = 16
NW = NUM_CORES * NUM_SUBCORES
B_PER_W = BATCH // NW
HALF = B_PER_W // 2


def _sbpr_body(idx_hbm, user_tab, item_tab,
               out_u, out_p, out_n, idx_v, rows_v, sem0, sem1, sem2):
    wid = lax.axis_index("s") * NUM_CORES + lax.axis_index("c")
    base = wid * B_PER_W

    pltpu.sync_copy(idx_hbm.at[wid], idx_v.at[pl.ds(0, 3 * B_PER_W)])

    tabs = (user_tab, item_tab, item_tab)
    sems = (sem0, sem1, sem2)
    outs = (out_u, out_p, out_n)

    for h in range(2):
        for k in range(3):
            def issue(j, carry, tab=tabs[k], sem=sems[k], k=k, h=h):
                row = idx_v[pl.ds(k * B_PER_W + h * HALF + j, 16)][0]
                pltpu.async_copy(tab.at[pl.ds(row, 1)], rows_v.at[k, pl.ds(j, 1)], sem)
                return carry

            lax.fori_loop(0, HALF, issue, 0)

        for k in range(3):
            pltpu.make_async_copy(
                tabs[k].at[pl.ds(0, HALF)], rows_v.at[k], sems[k]
            ).wait()
            pltpu.sync_copy(
                rows_v.at[k],
                outs[k].at[pl.ds(base + h * HALF, HALF)],
            )


@jax.jit
def _sbpr(idx_all, embed_user, embed_item):
    out = jax.ShapeDtypeStruct((BATCH, EMBED), jnp.float32)
    mesh = plsc.VectorSubcoreMesh(core_axis_name="c", subcore_axis_name="s")
    return pl.kernel(
        _sbpr_body,
        out_type=(out, out, out),
        mesh=mesh,
        scratch_types=[
            pltpu.VMEM((3 * B_PER_W + 16,), jnp.int32),
            pltpu.VMEM((3, HALF, EMBED), jnp.float32),
            pltpu.SemaphoreType.DMA,
            pltpu.SemaphoreType.DMA,
            pltpu.SemaphoreType.DMA,
        ],
    )(idx_all, embed_user, embed_item)


def kernel(batch_user, batch_pos_item, batch_neg_item, embed_user, embed_item):
    idx_all = (
        jnp.stack([batch_user, batch_pos_item, batch_neg_item])
        .reshape(3, NW, B_PER_W)
        .transpose(1, 0, 2)
        .reshape(NW, 3 * B_PER_W)
    )
    return _sbpr(idx_all, embed_user, embed_item)
